# baseline (speedup 1.0000x reference)
import functools

import jax
import jax.numpy as jnp
from jax import lax
from jax.experimental import pallas as pl
from jax.experimental.pallas import tpu as pltpu
from jax.experimental.pallas import tpu_sc as plsc

NB_VECS = 1000000
NB_DIMS = 64
BATCH = 16384
PAIR = 2

NC = 2
NS = 16
NW = NC * NS
LANES = 16

ROWS_PER_W = BATCH * PAIR // NW
PAIRS_PER_W = BATCH // NW
GCHUNK = 128
NCHUNK = ROWS_PER_W // GCHUNK
DCHUNK = NB_DIMS // LANES


def _sgns(vii_r, W):
    mesh = plsc.VectorSubcoreMesh(core_axis_name="c", subcore_axis_name="s")

    @functools.partial(
        pl.kernel,
        out_type=jax.ShapeDtypeStruct((BATCH,), jnp.float32),
        mesh=mesh,
        compiler_params=pltpu.CompilerParams(use_tc_tiling_on_sc=True),
        scratch_types=[
            pltpu.VMEM((PAIRS_PER_W,), jnp.int32),
            pltpu.VMEM((PAIRS_PER_W,), jnp.int32),
            pltpu.VMEM((2, GCHUNK, NB_DIMS), jnp.float32),
            pltpu.VMEM((PAIRS_PER_W,), jnp.float32),
            pltpu.SemaphoreType.DMA,
            pltpu.SemaphoreType.DMA,
        ],
    )
    def k(vii_hbm, w_hbm, out_hbm, ia_v, ib_v, rows_v, out_v, sem0,
          sem1):
        sems = (sem0, sem1)
        wid = lax.axis_index("c") * NS + lax.axis_index("s")

        base = pl.multiple_of(wid * PAIRS_PER_W, PAIRS_PER_W)
        pltpu.sync_copy(vii_hbm.at[0, pl.ds(base, PAIRS_PER_W)], ia_v)
        pltpu.sync_copy(vii_hbm.at[1, pl.ds(base, PAIRS_PER_W)], ib_v)

        lane = lax.iota(jnp.int32, LANES)
        bfly = [lane ^ (1 << s) for s in range(4)]

        def hsum(v):
            for idx in bfly:
                v = v + jnp.take(v, idx)
            return v

        def fire(j):
            buf = rows_v.at[j % 2]

            def body(g, _):
                off = j * (GCHUNK // 2) + g * LANES
                iva = ia_v[pl.ds(off, LANES)]
                ivb = ib_v[pl.ds(off, LANES)]
                for kk in range(LANES):
                    pltpu.async_copy(
                        w_hbm.at[pl.ds(iva[kk], 1), :],
                        buf.at[pl.ds(2 * (g * LANES + kk), 1), :],
                        sems[j % 2],
                    )
                    pltpu.async_copy(
                        w_hbm.at[pl.ds(ivb[kk], 1), :],
                        buf.at[pl.ds(2 * (g * LANES + kk) + 1, 1), :],
                        sems[j % 2],
                    )
                return 0

            lax.fori_loop(0, GCHUNK // 2 // LANES, body, 0)

        def drain(j):
            pltpu.make_async_copy(
                w_hbm.at[pl.ds(0, GCHUNK), :],
                rows_v.at[j % 2],
                sems[j % 2],
            ).wait()

        def compute(j):
            buf = rows_v.at[j % 2]

            def group_body(g, _):
                res = jnp.zeros((LANES,), jnp.float32)
                for jj in range(LANES):
                    i = g * LANES + jj
                    acc = (buf[2 * i, pl.ds(0, LANES)]
                           * buf[2 * i + 1, pl.ds(0, LANES)])
                    for kk in range(1, DCHUNK):
                        acc = acc + (buf[2 * i, pl.ds(kk * LANES, LANES)]
                                     * buf[2 * i + 1, pl.ds(kk * LANES,
                                                            LANES)])
                    res = jnp.where(lane == jj, hsum(acc), res)
                out_v[pl.ds(j * (GCHUNK // 2) + g * LANES, LANES)] = res
                return 0

            lax.fori_loop(0, GCHUNK // 2 // LANES, group_body, 0)

        fire(0)
        for j in range(1, NCHUNK):
            fire(j)
            drain(j - 1)
            compute(j - 1)
        drain(NCHUNK - 1)
        compute(NCHUNK - 1)

        pltpu.sync_copy(out_v, out_hbm.at[pl.ds(wid * PAIRS_PER_W,
                                                PAIRS_PER_W)])

    return k(vii_r, W)


def kernel(vii, W):
    vii_t = vii.astype(jnp.int32).T
    return _sgns(vii_t, W)
